# baseline (speedup 1.0000x reference)
import jax
import jax.numpy as jnp
from jax import lax
from jax.experimental import pallas as pl
from jax.experimental.pallas import tpu as pltpu
from jax.experimental.pallas import tpu_sc as plsc

NN = 50000
EE = 800000
EPAD = 819200
EROWS = EPAD // 128
NPAD = NN + 128
NGROUPS = 64
BLK = 1000
GRID = NN // BLK
RPT = 3128
RPT_LAST = NN - 15 * RPT


def _row_split_copy(s, src_fn, dst_fn):
    @pl.when(s < 15)
    def _():
        pltpu.sync_copy(src_fn(RPT), dst_fn(RPT))

    @pl.when(s == 15)
    def _():
        pltpu.sync_copy(src_fn(RPT_LAST), dst_fn(RPT_LAST))


_SC_MESH = dict(core_axis_name="c", subcore_axis_name="s")


def _deg_body(dst2_hbm, out_hbm, ones_v, didx_v, acc_s):
    c = lax.axis_index("c")
    s = lax.axis_index("s")

    def fill(i, carry):
        ones_v[i, pl.ds(0, 16)] = jnp.full((16,), 1.0, jnp.float32)
        return carry

    lax.fori_loop(0, RPT, fill, 0, unroll=4)
    r0 = s * RPT
    _row_split_copy(s, lambda n: ones_v.at[pl.ds(0, n)],
                    lambda n: acc_s.at[pl.ds(r0, n)])
    plsc.subcore_barrier()

    def blk(b, carry):
        er0 = c * 3200 + s * 200 + b * 8
        pltpu.sync_copy(dst2_hbm.at[pl.ds(er0, 8)], didx_v)
        for j in range(8):
            pltpu.sync_copy(ones_v.at[pl.ds(0, 128)], acc_s.at[didx_v.at[j]],
                            add=True)
        return carry

    lax.fori_loop(0, 25, blk, 0)
    plsc.subcore_barrier()
    _row_split_copy(s, lambda n: acc_s.at[pl.ds(r0, n)],
                    lambda n: out_hbm.at[pl.ds(c * NN + r0, n)])


_deg_call = pl.kernel(
    _deg_body,
    out_type=jax.ShapeDtypeStruct((2 * NN, 16), jnp.float32),
    mesh=plsc.VectorSubcoreMesh(**_SC_MESH),
    compiler_params=pltpu.CompilerParams(use_tc_tiling_on_sc=False),
    scratch_types=[
        pltpu.VMEM((RPT, 16), jnp.float32),
        pltpu.VMEM((8, 128), jnp.int32),
        pltpu.VMEM_SHARED((NPAD, 16), jnp.float32),
    ],
)


def _make_spmm(S):
    S_pc = S // 2

    def body(*refs):
        us = refs[:S]
        src2_hbm = refs[S]
        dst2_hbm = refs[S + 1]
        outs = refs[S + 2:2 * S + 2]
        sidx_v, didx_v, rows_v, isem, sems, acc_s = refs[2 * S + 2:]
        c = lax.axis_index("c")
        s = lax.axis_index("s")

        def slice_body(u_hbm, out_hbm):
            r0 = s * RPT
            _row_split_copy(s, lambda n: u_hbm.at[pl.ds(r0, n)],
                            lambda n: acc_s.at[pl.ds(r0, n)])
            plsc.subcore_barrier()

            row0 = s * 400
            pltpu.sync_copy(src2_hbm.at[pl.ds(row0, 16)],
                            sidx_v.at[pl.ds(0, 16)])
            pltpu.sync_copy(dst2_hbm.at[pl.ds(row0, 16)],
                            didx_v.at[pl.ds(0, 16)])

            def blk(b, bcarry):
                cur = lax.rem(b, 2) * 16
                nxt = lax.rem(b + 1, 2) * 16

                @pl.when(b > 0)
                def _():
                    pltpu.make_async_copy(
                        src2_hbm.at[pl.ds(row0 + b * 16, 16)],
                        sidx_v.at[pl.ds(cur, 16)], isem).wait()
                    pltpu.make_async_copy(
                        dst2_hbm.at[pl.ds(row0 + b * 16, 16)],
                        didx_v.at[pl.ds(cur, 16)], isem).wait()

                @pl.when(b < 24)
                def _():
                    pltpu.async_copy(
                        src2_hbm.at[pl.ds(row0 + (b + 1) * 16, 16)],
                        sidx_v.at[pl.ds(nxt, 16)], isem)
                    pltpu.async_copy(
                        dst2_hbm.at[pl.ds(row0 + (b + 1) * 16, 16)],
                        didx_v.at[pl.ds(nxt, 16)], isem)

                cps = [
                    pltpu.async_copy(u_hbm.at[sidx_v.at[cur + j]],
                                     rows_v.at[j], sems.at[j])
                    for j in range(4)
                ]
                for j in range(16):
                    jb = j % 4
                    cps[jb].wait()
                    pltpu.sync_copy(rows_v.at[jb],
                                    acc_s.at[didx_v.at[cur + j]], add=True)
                    if j + 4 < 16:
                        cps[jb] = pltpu.async_copy(
                            u_hbm.at[sidx_v.at[cur + j + 4]], rows_v.at[jb],
                            sems.at[jb])
                return bcarry

            lax.fori_loop(0, 25, blk, 0)
            plsc.subcore_barrier()
            _row_split_copy(s, lambda n: acc_s.at[pl.ds(r0, n)],
                            lambda n: out_hbm.at[pl.ds(r0, n)])
            plsc.subcore_barrier()

        for sl in range(S):
            owner = sl // S_pc

            @pl.when(c == owner)
            def _(u_ref=us[sl], o_ref=outs[sl]):
                slice_body(u_ref, o_ref)

    return pl.kernel(
        body,
        out_type=[jax.ShapeDtypeStruct((NN, 32), jnp.float32)
                  for _ in range(S)],
        mesh=plsc.VectorSubcoreMesh(**_SC_MESH),
        compiler_params=pltpu.CompilerParams(use_tc_tiling_on_sc=False),
        scratch_types=[
            pltpu.VMEM((32, 128), jnp.int32),
            pltpu.VMEM((32, 128), jnp.int32),
            pltpu.VMEM((4, 128, 32), jnp.float32),
            pltpu.SemaphoreType.DMA,
            pltpu.SemaphoreType.DMA((4,)),
            pltpu.VMEM_SHARED((NPAD, 32), jnp.float32),
        ],
    )


_spmm2 = _make_spmm(2)
_spmm4 = _make_spmm(4)

RPW = 1568
RBLK = 224
RPW_LAST = NN - 31 * RPW
POOL_TAIL = RPW_LAST - 6 * RBLK


def _pool_body(h_hbm, b_hbm, out_hbm, rows_v, bid_v, pacc_v):
    c = lax.axis_index("c")
    s = lax.axis_index("s")
    w = s * 2 + c

    def initp(i, carry):
        for j in range(4):
            pacc_v[i, pl.ds(j * 16, 16)] = jnp.full((16,), -jnp.inf,
                                                    jnp.float32)
        return carry

    lax.fori_loop(0, NGROUPS, initp, 0, unroll=4)

    def do_block(bb, nrows):
        pltpu.sync_copy(h_hbm.at[pl.ds(w * RPW + bb * RBLK, nrows)],
                        rows_v.at[pl.ds(0, nrows)])

        def row(r, rcarry):
            gid = bid_v[pl.ds(bb * RBLK + r, 16)][0]
            for j in range(4):
                sl_ = pl.ds(j * 16, 16)
                pacc_v[gid, sl_] = jnp.maximum(pacc_v[gid, sl_],
                                               rows_v[r, sl_])
            return rcarry

        lax.fori_loop(0, nrows, row, 0)

    @pl.when(w < 31)
    def _():
        pltpu.sync_copy(b_hbm.at[pl.ds(w * RPW, RPW)],
                        bid_v.at[pl.ds(0, RPW)])

        def blk(bb, carry):
            do_block(bb, RBLK)
            return carry

        lax.fori_loop(0, 7, blk, 0)

    @pl.when(w == 31)
    def _():
        pltpu.sync_copy(b_hbm.at[pl.ds(w * RPW, RPW_LAST)],
                        bid_v.at[pl.ds(0, RPW_LAST)])

        def blk(bb, carry):
            do_block(bb, RBLK)
            return carry

        lax.fori_loop(0, 6, blk, 0)
        do_block(6, POOL_TAIL)

    pltpu.sync_copy(pacc_v, out_hbm.at[w])


_pool_call = pl.kernel(
    _pool_body,
    out_type=jax.ShapeDtypeStruct((32, NGROUPS, 64), jnp.float32),
    mesh=plsc.VectorSubcoreMesh(**_SC_MESH),
    compiler_params=pltpu.CompilerParams(use_tc_tiling_on_sc=False),
    scratch_types=[
        pltpu.VMEM((RBLK, 64), jnp.float32),
        pltpu.VMEM((RPW + 16,), jnp.int32),
        pltpu.VMEM((NGROUPS, 64), jnp.float32),
    ],
)


def _gelu_f(x):
    return x * 0.5 * (1.0 + lax.erf(x * 0.7071067811865476))


def _dinv_body(d0_ref, d1_ref, o_ref):
    d = d0_ref[:, 0:1] + d1_ref[:, 0:1] - 1.0
    o_ref[...] = lax.rsqrt(d)


def _dinv_call(deg_parts):
    nb = NN // BLK
    return pl.pallas_call(
        _dinv_body,
        grid=(GRID,),
        in_specs=[
            pl.BlockSpec((BLK, 16), lambda i: (i, 0)),
            pl.BlockSpec((BLK, 16), lambda i: (nb + i, 0)),
        ],
        out_specs=pl.BlockSpec((BLK, 1), lambda i: (i, 0)),
        out_shape=jax.ShapeDtypeStruct((NN, 1), jnp.float32),
    )(deg_parts, deg_parts)


def _slice_outspecs(S_out):
    specs = [pl.BlockSpec((BLK, 32), lambda i: (i, 0)) for _ in range(S_out)]
    shapes = [jax.ShapeDtypeStruct((NN, 32), jnp.float32)
              for _ in range(S_out)]
    return specs, shapes


def _write_slices(u, o_refs):
    for i, o in enumerate(o_refs):
        o[...] = u[:, 32 * i:32 * (i + 1)]


def _pre1_body(x_ref, w_ref, dv_ref, *o_refs):
    u = jnp.dot(x_ref[...], w_ref[...],
                preferred_element_type=jnp.float32) * dv_ref[...]
    _write_slices(u, o_refs)


def _pre1_call(x, W1, dinv):
    o_specs, o_shapes = _slice_outspecs(2)
    return pl.pallas_call(
        _pre1_body,
        grid=(GRID,),
        in_specs=[
            pl.BlockSpec((BLK, 2), lambda i: (i, 0)),
            pl.BlockSpec((2, 64), lambda i: (0, 0)),
            pl.BlockSpec((BLK, 1), lambda i: (i, 0)),
        ],
        out_specs=o_specs,
        out_shape=o_shapes,
    )(x, W1, dinv)


def _make_post_pre(S_in, S_out, d_in, d_out):
    def body(*refs):
        a_refs = refs[:S_in]
        b_ref, dv_ref, w_ref = refs[S_in:S_in + 3]
        o_refs = refs[S_in + 3:]
        acc = jnp.concatenate([a[...] for a in a_refs], axis=1)
        dv = dv_ref[...]
        h = _gelu_f(dv * acc + b_ref[...])
        u = jnp.dot(h, w_ref[...], preferred_element_type=jnp.float32) * dv
        _write_slices(u, o_refs)

    def call(a_list, b, dinv, W):
        o_specs, o_shapes = _slice_outspecs(S_out)
        return pl.pallas_call(
            body,
            grid=(GRID,),
            in_specs=(
                [pl.BlockSpec((BLK, 32), lambda i: (i, 0))
                 for _ in range(S_in)] + [
                    pl.BlockSpec((1, d_in), lambda i: (0, 0)),
                    pl.BlockSpec((BLK, 1), lambda i: (i, 0)),
                    pl.BlockSpec((d_in, d_out), lambda i: (0, 0)),
                ]),
            out_specs=o_specs,
            out_shape=o_shapes,
        )(*a_list, b.reshape(1, d_in), dinv, W)

    return call


def _make_bnstat(S_in, d):
    def body(*refs):
        a_refs = refs[:S_in]
        b_ref, dv_ref = refs[S_in:S_in + 2]
        z_ref, s_ref, q_ref = refs[S_in + 2:]
        acc = jnp.concatenate([a[...] for a in a_refs], axis=1)
        z = dv_ref[...] * acc + b_ref[...]
        z_ref[...] = z
        s_ref[0] = jnp.sum(z, axis=0, keepdims=True)
        q_ref[0] = jnp.sum(z * z, axis=0, keepdims=True)

    def call(a_list, b, dinv):
        return pl.pallas_call(
            body,
            grid=(GRID,),
            in_specs=(
                [pl.BlockSpec((BLK, 32), lambda i: (i, 0))
                 for _ in range(S_in)] + [
                    pl.BlockSpec((1, d), lambda i: (0, 0)),
                    pl.BlockSpec((BLK, 1), lambda i: (i, 0)),
                ]),
            out_specs=[
                pl.BlockSpec((BLK, d), lambda i: (i, 0)),
                pl.BlockSpec((1, 1, d), lambda i: (i, 0, 0)),
                pl.BlockSpec((1, 1, d), lambda i: (i, 0, 0)),
            ],
            out_shape=[
                jax.ShapeDtypeStruct((NN, d), jnp.float32),
                jax.ShapeDtypeStruct((GRID, 1, d), jnp.float32),
                jax.ShapeDtypeStruct((GRID, 1, d), jnp.float32),
            ],
        )(*a_list, b.reshape(1, d), dinv)

    return call


def _make_bnapply(d, S_out, d_out, with_pre):
    def body(z_ref, s_ref, q_ref, g_ref, be_ref, *rest):
        m = jnp.sum(s_ref[...], axis=0) * (1.0 / NN)
        v = jnp.sum(q_ref[...], axis=0) * (1.0 / NN) - m * m
        inv = lax.rsqrt(v + 1e-5)
        h = _gelu_f((z_ref[...] - m) * inv * g_ref[...] + be_ref[...])
        if with_pre:
            w_ref, dv_ref = rest[:2]
            o_refs = rest[2:]
            u = jnp.dot(h, w_ref[...],
                        preferred_element_type=jnp.float32) * dv_ref[...]
            _write_slices(u, o_refs)
        else:
            (o_ref,) = rest
            o_ref[...] = h

    def call(z, sums, sq, g, be, W=None, dinv=None):
        in_specs = [
            pl.BlockSpec((BLK, d), lambda i: (i, 0)),
            pl.BlockSpec((GRID, 1, d), lambda i: (0, 0, 0)),
            pl.BlockSpec((GRID, 1, d), lambda i: (0, 0, 0)),
            pl.BlockSpec((1, d), lambda i: (0, 0)),
            pl.BlockSpec((1, d), lambda i: (0, 0)),
        ]
        args = [z, sums, sq, g.reshape(1, d), be.reshape(1, d)]
        if with_pre:
            in_specs += [
                pl.BlockSpec((d, d_out), lambda i: (0, 0)),
                pl.BlockSpec((BLK, 1), lambda i: (i, 0)),
            ]
            args += [W, dinv]
            out_specs, out_shape = _slice_outspecs(S_out)
        else:
            out_specs = pl.BlockSpec((BLK, d), lambda i: (i, 0))
            out_shape = jax.ShapeDtypeStruct((NN, d), jnp.float32)
        return pl.pallas_call(
            body,
            grid=(GRID,),
            in_specs=in_specs,
            out_specs=out_specs,
            out_shape=out_shape,
        )(*args)

    return call


def _head_body(p_ref, w1_ref, b1_ref, w2_ref, b2_ref, o_ref):
    pooled = jnp.max(p_ref[...], axis=0)
    t = jnp.dot(pooled, w1_ref[...],
                preferred_element_type=jnp.float32) + b1_ref[...]
    o_ref[...] = jnp.dot(t, w2_ref[...],
                         preferred_element_type=jnp.float32) + b2_ref[...]


def _head_call(parts, lin1_W, lin1_b, lin_W, lin_b):
    return pl.pallas_call(
        _head_body,
        out_shape=jax.ShapeDtypeStruct((NGROUPS, 2), jnp.float32),
    )(parts, lin1_W, lin1_b.reshape(1, 10), lin_W, lin_b.reshape(1, 2))


_bnstat64 = _make_bnstat(2, 64)
_bnstat128 = _make_bnstat(4, 128)
_bnapply64_pre = _make_bnapply(64, 2, 64, True)
_bnapply128_pre = _make_bnapply(128, 4, 128, True)
_bnapply64_out = _make_bnapply(64, 0, 0, False)
_post64_128 = _make_post_pre(2, 4, 64, 128)
_post128_64 = _make_post_pre(4, 2, 128, 64)


def kernel(x, ei, batch, W1, b1, W2, b2, W3, b3, W4, b4, W5, b5, g1, be1, g2,
           be2, g3, be3, lin1_W, lin1_b, lin_W, lin_b):
    f32 = jnp.float32
    src = ei[0]
    dst = ei[1]
    pad_idx = jnp.arange(EPAD - EE, dtype=jnp.int32) % 128
    src_p = jnp.concatenate([src, pad_idx]).reshape(EROWS, 128)
    dst_p = jnp.concatenate([dst, pad_idx + NN]).reshape(EROWS, 128)

    deg_parts = _deg_call(dst_p)
    dinv = _dinv_call(deg_parts)

    u1 = _pre1_call(x.astype(f32), W1, dinv)
    a1 = _spmm2(*u1, src_p, dst_p)
    z1, s1, q1 = _bnstat64(a1, b1, dinv)
    u2 = _bnapply64_pre(z1, s1, q1, g1, be1, W2, dinv)
    a2 = _spmm2(*u2, src_p, dst_p)
    u3 = _post64_128(a2, b2, dinv, W3)
    a3 = _spmm4(*u3, src_p, dst_p)
    z3, s3, q3 = _bnstat128(a3, b3, dinv)
    u4 = _bnapply128_pre(z3, s3, q3, g2, be2, W4, dinv)
    a4 = _spmm4(*u4, src_p, dst_p)
    u5 = _post128_64(a4, b4, dinv, W5)
    a5 = _spmm2(*u5, src_p, dst_p)
    z5, s5, q5 = _bnstat64(a5, b5, dinv)
    h5 = _bnapply64_out(z5, s5, q5, g3, be3)

    parts = _pool_call(h5, batch)
    return _head_call(parts, lin1_W, lin1_b, lin_W, lin_b)


# speedup vs baseline: 1.0898x; 1.0898x over previous
import jax
import jax.numpy as jnp
from jax import lax
from jax.experimental import pallas as pl
from jax.experimental.pallas import tpu as pltpu
from jax.experimental.pallas import tpu_sc as plsc

NN = 50000
EE = 800000
EPAD = 819200
EROWS = EPAD // 128
NPAD = NN + 128
NGROUPS = 64
BLK = 5000
GRID = NN // BLK
RPT = 3128
RPT_LAST = NN - 15 * RPT


def _row_split_copy(s, src_fn, dst_fn):
    @pl.when(s < 15)
    def _():
        pltpu.sync_copy(src_fn(RPT), dst_fn(RPT))

    @pl.when(s == 15)
    def _():
        pltpu.sync_copy(src_fn(RPT_LAST), dst_fn(RPT_LAST))


_SC_MESH = dict(core_axis_name="c", subcore_axis_name="s")


def _deg_body(dst2_hbm, out_hbm, ones_v, didx_v, acc_s):
    c = lax.axis_index("c")
    s = lax.axis_index("s")

    def fill(i, carry):
        ones_v[i, pl.ds(0, 16)] = jnp.full((16,), 1.0, jnp.float32)
        return carry

    lax.fori_loop(0, RPT, fill, 0, unroll=4)
    r0 = s * RPT
    _row_split_copy(s, lambda n: ones_v.at[pl.ds(0, n)],
                    lambda n: acc_s.at[pl.ds(r0, n)])
    plsc.subcore_barrier()

    def blk(b, carry):
        er0 = c * 3200 + s * 200 + b * 8
        pltpu.sync_copy(dst2_hbm.at[pl.ds(er0, 8)], didx_v)
        for j in range(8):
            pltpu.sync_copy(ones_v.at[pl.ds(0, 128)], acc_s.at[didx_v.at[j]],
                            add=True)
        return carry

    lax.fori_loop(0, 25, blk, 0)
    plsc.subcore_barrier()
    _row_split_copy(s, lambda n: acc_s.at[pl.ds(r0, n)],
                    lambda n: out_hbm.at[pl.ds(c * NN + r0, n)])


_deg_call = pl.kernel(
    _deg_body,
    out_type=jax.ShapeDtypeStruct((2 * NN, 16), jnp.float32),
    mesh=plsc.VectorSubcoreMesh(**_SC_MESH),
    compiler_params=pltpu.CompilerParams(use_tc_tiling_on_sc=False),
    scratch_types=[
        pltpu.VMEM((RPT, 16), jnp.float32),
        pltpu.VMEM((8, 128), jnp.int32),
        pltpu.VMEM_SHARED((NPAD, 16), jnp.float32),
    ],
)


def _make_spmm(S):
    S_pc = S // 2

    def body(*refs):
        us = refs[:S]
        src2_hbm = refs[S]
        dst2_hbm = refs[S + 1]
        outs = refs[S + 2:2 * S + 2]
        sidx_v, didx_v, rows_v, isem, sems, acc_s = refs[2 * S + 2:]
        c = lax.axis_index("c")
        s = lax.axis_index("s")

        def slice_body(u_hbm, out_hbm):
            r0 = s * RPT
            _row_split_copy(s, lambda n: u_hbm.at[pl.ds(r0, n)],
                            lambda n: acc_s.at[pl.ds(r0, n)])
            plsc.subcore_barrier()

            row0 = s * 400
            pltpu.sync_copy(src2_hbm.at[pl.ds(row0, 16)],
                            sidx_v.at[pl.ds(0, 16)])
            pltpu.sync_copy(dst2_hbm.at[pl.ds(row0, 16)],
                            didx_v.at[pl.ds(0, 16)])

            def blk(b, bcarry):
                cur = lax.rem(b, 2) * 16
                nxt = lax.rem(b + 1, 2) * 16

                @pl.when(b > 0)
                def _():
                    pltpu.make_async_copy(
                        src2_hbm.at[pl.ds(row0 + b * 16, 16)],
                        sidx_v.at[pl.ds(cur, 16)], isem).wait()
                    pltpu.make_async_copy(
                        dst2_hbm.at[pl.ds(row0 + b * 16, 16)],
                        didx_v.at[pl.ds(cur, 16)], isem).wait()

                @pl.when(b < 24)
                def _():
                    pltpu.async_copy(
                        src2_hbm.at[pl.ds(row0 + (b + 1) * 16, 16)],
                        sidx_v.at[pl.ds(nxt, 16)], isem)
                    pltpu.async_copy(
                        dst2_hbm.at[pl.ds(row0 + (b + 1) * 16, 16)],
                        didx_v.at[pl.ds(nxt, 16)], isem)

                cps = [
                    pltpu.async_copy(u_hbm.at[sidx_v.at[cur + j]],
                                     rows_v.at[j], sems.at[j])
                    for j in range(4)
                ]
                for j in range(16):
                    jb = j % 4
                    cps[jb].wait()
                    pltpu.sync_copy(rows_v.at[jb],
                                    acc_s.at[didx_v.at[cur + j]], add=True)
                    if j + 4 < 16:
                        cps[jb] = pltpu.async_copy(
                            u_hbm.at[sidx_v.at[cur + j + 4]], rows_v.at[jb],
                            sems.at[jb])
                return bcarry

            lax.fori_loop(0, 25, blk, 0)
            plsc.subcore_barrier()
            _row_split_copy(s, lambda n: acc_s.at[pl.ds(r0, n)],
                            lambda n: out_hbm.at[pl.ds(r0, n)])
            plsc.subcore_barrier()

        for sl in range(S):
            owner = sl // S_pc

            @pl.when(c == owner)
            def _(u_ref=us[sl], o_ref=outs[sl]):
                slice_body(u_ref, o_ref)

    return pl.kernel(
        body,
        out_type=[jax.ShapeDtypeStruct((NN, 32), jnp.float32)
                  for _ in range(S)],
        mesh=plsc.VectorSubcoreMesh(**_SC_MESH),
        compiler_params=pltpu.CompilerParams(use_tc_tiling_on_sc=False),
        scratch_types=[
            pltpu.VMEM((32, 128), jnp.int32),
            pltpu.VMEM((32, 128), jnp.int32),
            pltpu.VMEM((4, 128, 32), jnp.float32),
            pltpu.SemaphoreType.DMA,
            pltpu.SemaphoreType.DMA((4,)),
            pltpu.VMEM_SHARED((NPAD, 32), jnp.float32),
        ],
    )


_spmm2 = _make_spmm(2)
_spmm4 = _make_spmm(4)

RPW = 1568
RBLK = 224
RPW_LAST = NN - 31 * RPW
POOL_TAIL = RPW_LAST - 6 * RBLK


def _pool_body(h_hbm, b_hbm, out_hbm, rows_v, bid_v, pacc_v):
    c = lax.axis_index("c")
    s = lax.axis_index("s")
    w = s * 2 + c

    def initp(i, carry):
        for j in range(4):
            pacc_v[i, pl.ds(j * 16, 16)] = jnp.full((16,), -jnp.inf,
                                                    jnp.float32)
        return carry

    lax.fori_loop(0, NGROUPS, initp, 0, unroll=4)

    def do_block(bb, nrows):
        pltpu.sync_copy(h_hbm.at[pl.ds(w * RPW + bb * RBLK, nrows)],
                        rows_v.at[pl.ds(0, nrows)])

        def row(r, rcarry):
            gid = bid_v[pl.ds(bb * RBLK + r, 16)][0]
            for j in range(4):
                sl_ = pl.ds(j * 16, 16)
                pacc_v[gid, sl_] = jnp.maximum(pacc_v[gid, sl_],
                                               rows_v[r, sl_])
            return rcarry

        lax.fori_loop(0, nrows, row, 0)

    @pl.when(w < 31)
    def _():
        pltpu.sync_copy(b_hbm.at[pl.ds(w * RPW, RPW)],
                        bid_v.at[pl.ds(0, RPW)])

        def blk(bb, carry):
            do_block(bb, RBLK)
            return carry

        lax.fori_loop(0, 7, blk, 0)

    @pl.when(w == 31)
    def _():
        pltpu.sync_copy(b_hbm.at[pl.ds(w * RPW, RPW_LAST)],
                        bid_v.at[pl.ds(0, RPW_LAST)])

        def blk(bb, carry):
            do_block(bb, RBLK)
            return carry

        lax.fori_loop(0, 6, blk, 0)
        do_block(6, POOL_TAIL)

    pltpu.sync_copy(pacc_v, out_hbm.at[w])


_pool_call = pl.kernel(
    _pool_body,
    out_type=jax.ShapeDtypeStruct((32, NGROUPS, 64), jnp.float32),
    mesh=plsc.VectorSubcoreMesh(**_SC_MESH),
    compiler_params=pltpu.CompilerParams(use_tc_tiling_on_sc=False),
    scratch_types=[
        pltpu.VMEM((RBLK, 64), jnp.float32),
        pltpu.VMEM((RPW + 16,), jnp.int32),
        pltpu.VMEM((NGROUPS, 64), jnp.float32),
    ],
)


def _gelu_f(x):
    return x * 0.5 * (1.0 + lax.erf(x * 0.7071067811865476))


def _dinv_body(d0_ref, d1_ref, o_ref):
    d = d0_ref[:, 0:1] + d1_ref[:, 0:1] - 1.0
    o_ref[...] = lax.rsqrt(d)


def _dinv_call(deg_parts):
    nb = NN // BLK
    return pl.pallas_call(
        _dinv_body,
        grid=(GRID,),
        in_specs=[
            pl.BlockSpec((BLK, 16), lambda i: (i, 0)),
            pl.BlockSpec((BLK, 16), lambda i: (nb + i, 0)),
        ],
        out_specs=pl.BlockSpec((BLK, 1), lambda i: (i, 0)),
        out_shape=jax.ShapeDtypeStruct((NN, 1), jnp.float32),
    )(deg_parts, deg_parts)


def _slice_outspecs(S_out):
    specs = [pl.BlockSpec((BLK, 32), lambda i: (i, 0)) for _ in range(S_out)]
    shapes = [jax.ShapeDtypeStruct((NN, 32), jnp.float32)
              for _ in range(S_out)]
    return specs, shapes


def _write_slices(u, o_refs):
    for i, o in enumerate(o_refs):
        o[...] = u[:, 32 * i:32 * (i + 1)]


def _pre1_body(x_ref, w_ref, dv_ref, *o_refs):
    u = jnp.dot(x_ref[...], w_ref[...],
                preferred_element_type=jnp.float32) * dv_ref[...]
    _write_slices(u, o_refs)


def _pre1_call(x, W1, dinv):
    o_specs, o_shapes = _slice_outspecs(2)
    return pl.pallas_call(
        _pre1_body,
        grid=(GRID,),
        in_specs=[
            pl.BlockSpec((BLK, 2), lambda i: (i, 0)),
            pl.BlockSpec((2, 64), lambda i: (0, 0)),
            pl.BlockSpec((BLK, 1), lambda i: (i, 0)),
        ],
        out_specs=o_specs,
        out_shape=o_shapes,
    )(x, W1, dinv)


def _make_post_pre(S_in, S_out, d_in, d_out):
    def body(*refs):
        a_refs = refs[:S_in]
        b_ref, dv_ref, w_ref = refs[S_in:S_in + 3]
        o_refs = refs[S_in + 3:]
        acc = jnp.concatenate([a[...] for a in a_refs], axis=1)
        dv = dv_ref[...]
        h = _gelu_f(dv * acc + b_ref[...])
        u = jnp.dot(h, w_ref[...], preferred_element_type=jnp.float32) * dv
        _write_slices(u, o_refs)

    def call(a_list, b, dinv, W):
        o_specs, o_shapes = _slice_outspecs(S_out)
        return pl.pallas_call(
            body,
            grid=(GRID,),
            in_specs=(
                [pl.BlockSpec((BLK, 32), lambda i: (i, 0))
                 for _ in range(S_in)] + [
                    pl.BlockSpec((1, d_in), lambda i: (0, 0)),
                    pl.BlockSpec((BLK, 1), lambda i: (i, 0)),
                    pl.BlockSpec((d_in, d_out), lambda i: (0, 0)),
                ]),
            out_specs=o_specs,
            out_shape=o_shapes,
        )(*a_list, b.reshape(1, d_in), dinv, W)

    return call


def _make_bnstat(S_in, d):
    def body(*refs):
        a_refs = refs[:S_in]
        b_ref, dv_ref = refs[S_in:S_in + 2]
        z_ref, s_ref, q_ref = refs[S_in + 2:]
        acc = jnp.concatenate([a[...] for a in a_refs], axis=1)
        z = dv_ref[...] * acc + b_ref[...]
        z_ref[...] = z
        s_ref[0] = jnp.sum(z, axis=0, keepdims=True)
        q_ref[0] = jnp.sum(z * z, axis=0, keepdims=True)

    def call(a_list, b, dinv):
        return pl.pallas_call(
            body,
            grid=(GRID,),
            in_specs=(
                [pl.BlockSpec((BLK, 32), lambda i: (i, 0))
                 for _ in range(S_in)] + [
                    pl.BlockSpec((1, d), lambda i: (0, 0)),
                    pl.BlockSpec((BLK, 1), lambda i: (i, 0)),
                ]),
            out_specs=[
                pl.BlockSpec((BLK, d), lambda i: (i, 0)),
                pl.BlockSpec((1, 1, d), lambda i: (i, 0, 0)),
                pl.BlockSpec((1, 1, d), lambda i: (i, 0, 0)),
            ],
            out_shape=[
                jax.ShapeDtypeStruct((NN, d), jnp.float32),
                jax.ShapeDtypeStruct((GRID, 1, d), jnp.float32),
                jax.ShapeDtypeStruct((GRID, 1, d), jnp.float32),
            ],
        )(*a_list, b.reshape(1, d), dinv)

    return call


def _make_bnapply(d, S_out, d_out, with_pre):
    def body(z_ref, s_ref, q_ref, g_ref, be_ref, *rest):
        m = jnp.sum(s_ref[...], axis=0) * (1.0 / NN)
        v = jnp.sum(q_ref[...], axis=0) * (1.0 / NN) - m * m
        inv = lax.rsqrt(v + 1e-5)
        h = _gelu_f((z_ref[...] - m) * inv * g_ref[...] + be_ref[...])
        if with_pre:
            w_ref, dv_ref = rest[:2]
            o_refs = rest[2:]
            u = jnp.dot(h, w_ref[...],
                        preferred_element_type=jnp.float32) * dv_ref[...]
            _write_slices(u, o_refs)
        else:
            (o_ref,) = rest
            o_ref[...] = h

    def call(z, sums, sq, g, be, W=None, dinv=None):
        in_specs = [
            pl.BlockSpec((BLK, d), lambda i: (i, 0)),
            pl.BlockSpec((GRID, 1, d), lambda i: (0, 0, 0)),
            pl.BlockSpec((GRID, 1, d), lambda i: (0, 0, 0)),
            pl.BlockSpec((1, d), lambda i: (0, 0)),
            pl.BlockSpec((1, d), lambda i: (0, 0)),
        ]
        args = [z, sums, sq, g.reshape(1, d), be.reshape(1, d)]
        if with_pre:
            in_specs += [
                pl.BlockSpec((d, d_out), lambda i: (0, 0)),
                pl.BlockSpec((BLK, 1), lambda i: (i, 0)),
            ]
            args += [W, dinv]
            out_specs, out_shape = _slice_outspecs(S_out)
        else:
            out_specs = pl.BlockSpec((BLK, d), lambda i: (i, 0))
            out_shape = jax.ShapeDtypeStruct((NN, d), jnp.float32)
        return pl.pallas_call(
            body,
            grid=(GRID,),
            in_specs=in_specs,
            out_specs=out_specs,
            out_shape=out_shape,
        )(*args)

    return call


def _head_body(p_ref, w1_ref, b1_ref, w2_ref, b2_ref, o_ref):
    pooled = jnp.max(p_ref[...], axis=0)
    t = jnp.dot(pooled, w1_ref[...],
                preferred_element_type=jnp.float32) + b1_ref[...]
    o_ref[...] = jnp.dot(t, w2_ref[...],
                         preferred_element_type=jnp.float32) + b2_ref[...]


def _head_call(parts, lin1_W, lin1_b, lin_W, lin_b):
    return pl.pallas_call(
        _head_body,
        out_shape=jax.ShapeDtypeStruct((NGROUPS, 2), jnp.float32),
    )(parts, lin1_W, lin1_b.reshape(1, 10), lin_W, lin_b.reshape(1, 2))


_bnstat64 = _make_bnstat(2, 64)
_bnstat128 = _make_bnstat(4, 128)
_bnapply64_pre = _make_bnapply(64, 2, 64, True)
_bnapply128_pre = _make_bnapply(128, 4, 128, True)
_bnapply64_out = _make_bnapply(64, 0, 0, False)
_post64_128 = _make_post_pre(2, 4, 64, 128)
_post128_64 = _make_post_pre(4, 2, 128, 64)


def kernel(x, ei, batch, W1, b1, W2, b2, W3, b3, W4, b4, W5, b5, g1, be1, g2,
           be2, g3, be3, lin1_W, lin1_b, lin_W, lin_b):
    f32 = jnp.float32
    src = ei[0]
    dst = ei[1]
    pad_idx = jnp.arange(EPAD - EE, dtype=jnp.int32) % 128
    src_p = jnp.concatenate([src, pad_idx]).reshape(EROWS, 128)
    dst_p = jnp.concatenate([dst, pad_idx + NN]).reshape(EROWS, 128)

    deg_parts = _deg_call(dst_p)
    dinv = _dinv_call(deg_parts)

    u1 = _pre1_call(x.astype(f32), W1, dinv)
    a1 = _spmm2(*u1, src_p, dst_p)
    z1, s1, q1 = _bnstat64(a1, b1, dinv)
    u2 = _bnapply64_pre(z1, s1, q1, g1, be1, W2, dinv)
    a2 = _spmm2(*u2, src_p, dst_p)
    u3 = _post64_128(a2, b2, dinv, W3)
    a3 = _spmm4(*u3, src_p, dst_p)
    z3, s3, q3 = _bnstat128(a3, b3, dinv)
    u4 = _bnapply128_pre(z3, s3, q3, g2, be2, W4, dinv)
    a4 = _spmm4(*u4, src_p, dst_p)
    u5 = _post128_64(a4, b4, dinv, W5)
    a5 = _spmm2(*u5, src_p, dst_p)
    z5, s5, q5 = _bnstat64(a5, b5, dinv)
    h5 = _bnapply64_out(z5, s5, q5, g3, be3)

    parts = _pool_call(h5, batch)
    return _head_call(parts, lin1_W, lin1_b, lin_W, lin_b)


# speedup vs baseline: 1.1219x; 1.0294x over previous
import jax
import jax.numpy as jnp
from jax import lax
from jax.experimental import pallas as pl
from jax.experimental.pallas import tpu as pltpu
from jax.experimental.pallas import tpu_sc as plsc

NN = 50000
EE = 800000
EPAD = 819200
EROWS = EPAD // 128
NPAD = NN + 128
NGROUPS = 64
BLK = 5000
GRID = NN // BLK
RPT = 3128
RPT_LAST = NN - 15 * RPT


def _row_split_copy(s, src_fn, dst_fn):
    @pl.when(s < 15)
    def _():
        pltpu.sync_copy(src_fn(RPT), dst_fn(RPT))

    @pl.when(s == 15)
    def _():
        pltpu.sync_copy(src_fn(RPT_LAST), dst_fn(RPT_LAST))


_SC_MESH = dict(core_axis_name="c", subcore_axis_name="s")


def _deg_body(dst2_hbm, out_hbm, ones_v, didx_v, acc_s):
    c = lax.axis_index("c")
    s = lax.axis_index("s")

    def fill(i, carry):
        ones_v[i, pl.ds(0, 16)] = jnp.full((16,), 1.0, jnp.float32)
        return carry

    lax.fori_loop(0, RPT, fill, 0, unroll=4)
    r0 = s * RPT
    _row_split_copy(s, lambda n: ones_v.at[pl.ds(0, n)],
                    lambda n: acc_s.at[pl.ds(r0, n)])
    plsc.subcore_barrier()

    def blk(b, carry):
        er0 = c * 3200 + s * 200 + b * 8
        pltpu.sync_copy(dst2_hbm.at[pl.ds(er0, 8)], didx_v)
        for j in range(8):
            pltpu.sync_copy(ones_v.at[pl.ds(0, 128)], acc_s.at[didx_v.at[j]],
                            add=True)
        return carry

    lax.fori_loop(0, 25, blk, 0)
    plsc.subcore_barrier()
    _row_split_copy(s, lambda n: acc_s.at[pl.ds(r0, n)],
                    lambda n: out_hbm.at[pl.ds(c * NN + r0, n)])


_deg_call = pl.kernel(
    _deg_body,
    out_type=jax.ShapeDtypeStruct((2 * NN, 16), jnp.float32),
    mesh=plsc.VectorSubcoreMesh(**_SC_MESH),
    compiler_params=pltpu.CompilerParams(use_tc_tiling_on_sc=False),
    scratch_types=[
        pltpu.VMEM((RPT, 16), jnp.float32),
        pltpu.VMEM((8, 128), jnp.int32),
        pltpu.VMEM_SHARED((NPAD, 16), jnp.float32),
    ],
)


def _make_spmm(S):
    S_pc = S // 2

    def body(*refs):
        us = refs[:S]
        src2_hbm = refs[S]
        dst2_hbm = refs[S + 1]
        outs = refs[S + 2:2 * S + 2]
        sidx_v, didx_v, rows_v, isem, sems, acc_s = refs[2 * S + 2:]
        c = lax.axis_index("c")
        s = lax.axis_index("s")

        def slice_body(u_hbm, out_hbm):
            r0 = s * RPT
            _row_split_copy(s, lambda n: u_hbm.at[pl.ds(r0, n)],
                            lambda n: acc_s.at[pl.ds(r0, n)])
            plsc.subcore_barrier()

            row0 = s * 400
            pltpu.sync_copy(src2_hbm.at[pl.ds(row0, 16)],
                            sidx_v.at[pl.ds(0, 16)])
            pltpu.sync_copy(dst2_hbm.at[pl.ds(row0, 16)],
                            didx_v.at[pl.ds(0, 16)])

            def blk(b, bcarry):
                cur = lax.rem(b, 2) * 16
                nxt = lax.rem(b + 1, 2) * 16

                @pl.when(b > 0)
                def _():
                    pltpu.make_async_copy(
                        src2_hbm.at[pl.ds(row0 + b * 16, 16)],
                        sidx_v.at[pl.ds(cur, 16)], isem).wait()
                    pltpu.make_async_copy(
                        dst2_hbm.at[pl.ds(row0 + b * 16, 16)],
                        didx_v.at[pl.ds(cur, 16)], isem).wait()

                @pl.when(b < 24)
                def _():
                    pltpu.async_copy(
                        src2_hbm.at[pl.ds(row0 + (b + 1) * 16, 16)],
                        sidx_v.at[pl.ds(nxt, 16)], isem)
                    pltpu.async_copy(
                        dst2_hbm.at[pl.ds(row0 + (b + 1) * 16, 16)],
                        didx_v.at[pl.ds(nxt, 16)], isem)

                cps = [
                    pltpu.async_copy(u_hbm.at[sidx_v.at[cur + j]],
                                     rows_v.at[j], sems.at[j])
                    for j in range(5)
                ]
                for j in range(16):
                    jb = j % 5
                    cps[jb].wait()
                    pltpu.sync_copy(rows_v.at[jb],
                                    acc_s.at[didx_v.at[cur + j]], add=True)
                    if j + 5 < 16:
                        cps[jb] = pltpu.async_copy(
                            u_hbm.at[sidx_v.at[cur + j + 5]], rows_v.at[jb],
                            sems.at[jb])
                return bcarry

            lax.fori_loop(0, 25, blk, 0)
            plsc.subcore_barrier()
            _row_split_copy(s, lambda n: acc_s.at[pl.ds(r0, n)],
                            lambda n: out_hbm.at[pl.ds(r0, n)])
            plsc.subcore_barrier()

        for sl in range(S):
            owner = sl // S_pc

            @pl.when(c == owner)
            def _(u_ref=us[sl], o_ref=outs[sl]):
                slice_body(u_ref, o_ref)

    return pl.kernel(
        body,
        out_type=[jax.ShapeDtypeStruct((NN, 32), jnp.float32)
                  for _ in range(S)],
        mesh=plsc.VectorSubcoreMesh(**_SC_MESH),
        compiler_params=pltpu.CompilerParams(use_tc_tiling_on_sc=False),
        scratch_types=[
            pltpu.VMEM((32, 128), jnp.int32),
            pltpu.VMEM((32, 128), jnp.int32),
            pltpu.VMEM((5, 128, 32), jnp.float32),
            pltpu.SemaphoreType.DMA,
            pltpu.SemaphoreType.DMA((5,)),
            pltpu.VMEM_SHARED((NPAD, 32), jnp.float32),
        ],
    )


_spmm2 = _make_spmm(2)
_spmm4 = _make_spmm(4)

RPW = 1568
RBLK = 224
RPW_LAST = NN - 31 * RPW
POOL_TAIL = RPW_LAST - 6 * RBLK


def _pool_body(h_hbm, b_hbm, out_hbm, rows_v, bid_v, pacc_v):
    c = lax.axis_index("c")
    s = lax.axis_index("s")
    w = s * 2 + c

    def initp(i, carry):
        for j in range(4):
            pacc_v[i, pl.ds(j * 16, 16)] = jnp.full((16,), -jnp.inf,
                                                    jnp.float32)
        return carry

    lax.fori_loop(0, NGROUPS, initp, 0, unroll=4)

    def do_block(bb, nrows):
        pltpu.sync_copy(h_hbm.at[pl.ds(w * RPW + bb * RBLK, nrows)],
                        rows_v.at[pl.ds(0, nrows)])

        def row(r, rcarry):
            gid = bid_v[pl.ds(bb * RBLK + r, 16)][0]
            for j in range(4):
                sl_ = pl.ds(j * 16, 16)
                pacc_v[gid, sl_] = jnp.maximum(pacc_v[gid, sl_],
                                               rows_v[r, sl_])
            return rcarry

        lax.fori_loop(0, nrows, row, 0)

    @pl.when(w < 31)
    def _():
        pltpu.sync_copy(b_hbm.at[pl.ds(w * RPW, RPW)],
                        bid_v.at[pl.ds(0, RPW)])

        def blk(bb, carry):
            do_block(bb, RBLK)
            return carry

        lax.fori_loop(0, 7, blk, 0)

    @pl.when(w == 31)
    def _():
        pltpu.sync_copy(b_hbm.at[pl.ds(w * RPW, RPW_LAST)],
                        bid_v.at[pl.ds(0, RPW_LAST)])

        def blk(bb, carry):
            do_block(bb, RBLK)
            return carry

        lax.fori_loop(0, 6, blk, 0)
        do_block(6, POOL_TAIL)

    pltpu.sync_copy(pacc_v, out_hbm.at[w])


_pool_call = pl.kernel(
    _pool_body,
    out_type=jax.ShapeDtypeStruct((32, NGROUPS, 64), jnp.float32),
    mesh=plsc.VectorSubcoreMesh(**_SC_MESH),
    compiler_params=pltpu.CompilerParams(use_tc_tiling_on_sc=False),
    scratch_types=[
        pltpu.VMEM((RBLK, 64), jnp.float32),
        pltpu.VMEM((RPW + 16,), jnp.int32),
        pltpu.VMEM((NGROUPS, 64), jnp.float32),
    ],
)


def _gelu_f(x):
    return x * 0.5 * (1.0 + lax.erf(x * 0.7071067811865476))


def _dinv_body(d0_ref, d1_ref, o_ref):
    d = d0_ref[:, 0:1] + d1_ref[:, 0:1] - 1.0
    o_ref[...] = lax.rsqrt(d)


def _dinv_call(deg_parts):
    nb = NN // BLK
    return pl.pallas_call(
        _dinv_body,
        grid=(GRID,),
        in_specs=[
            pl.BlockSpec((BLK, 16), lambda i: (i, 0)),
            pl.BlockSpec((BLK, 16), lambda i: (nb + i, 0)),
        ],
        out_specs=pl.BlockSpec((BLK, 1), lambda i: (i, 0)),
        out_shape=jax.ShapeDtypeStruct((NN, 1), jnp.float32),
    )(deg_parts, deg_parts)


def _slice_outspecs(S_out):
    specs = [pl.BlockSpec((BLK, 32), lambda i: (i, 0)) for _ in range(S_out)]
    shapes = [jax.ShapeDtypeStruct((NN, 32), jnp.float32)
              for _ in range(S_out)]
    return specs, shapes


def _write_slices(u, o_refs):
    for i, o in enumerate(o_refs):
        o[...] = u[:, 32 * i:32 * (i + 1)]


def _pre1_body(x_ref, w_ref, dv_ref, *o_refs):
    u = jnp.dot(x_ref[...], w_ref[...],
                preferred_element_type=jnp.float32) * dv_ref[...]
    _write_slices(u, o_refs)


def _pre1_call(x, W1, dinv):
    o_specs, o_shapes = _slice_outspecs(2)
    return pl.pallas_call(
        _pre1_body,
        grid=(GRID,),
        in_specs=[
            pl.BlockSpec((BLK, 2), lambda i: (i, 0)),
            pl.BlockSpec((2, 64), lambda i: (0, 0)),
            pl.BlockSpec((BLK, 1), lambda i: (i, 0)),
        ],
        out_specs=o_specs,
        out_shape=o_shapes,
    )(x, W1, dinv)


def _make_post_pre(S_in, S_out, d_in, d_out):
    def body(*refs):
        a_refs = refs[:S_in]
        b_ref, dv_ref, w_ref = refs[S_in:S_in + 3]
        o_refs = refs[S_in + 3:]
        acc = jnp.concatenate([a[...] for a in a_refs], axis=1)
        dv = dv_ref[...]
        h = _gelu_f(dv * acc + b_ref[...])
        u = jnp.dot(h, w_ref[...], preferred_element_type=jnp.float32) * dv
        _write_slices(u, o_refs)

    def call(a_list, b, dinv, W):
        o_specs, o_shapes = _slice_outspecs(S_out)
        return pl.pallas_call(
            body,
            grid=(GRID,),
            in_specs=(
                [pl.BlockSpec((BLK, 32), lambda i: (i, 0))
                 for _ in range(S_in)] + [
                    pl.BlockSpec((1, d_in), lambda i: (0, 0)),
                    pl.BlockSpec((BLK, 1), lambda i: (i, 0)),
                    pl.BlockSpec((d_in, d_out), lambda i: (0, 0)),
                ]),
            out_specs=o_specs,
            out_shape=o_shapes,
        )(*a_list, b.reshape(1, d_in), dinv, W)

    return call


def _make_bnstat(S_in, d):
    def body(*refs):
        a_refs = refs[:S_in]
        b_ref, dv_ref = refs[S_in:S_in + 2]
        z_ref, s_ref, q_ref = refs[S_in + 2:]
        acc = jnp.concatenate([a[...] for a in a_refs], axis=1)
        z = dv_ref[...] * acc + b_ref[...]
        z_ref[...] = z
        s_ref[0] = jnp.sum(z, axis=0, keepdims=True)
        q_ref[0] = jnp.sum(z * z, axis=0, keepdims=True)

    def call(a_list, b, dinv):
        return pl.pallas_call(
            body,
            grid=(GRID,),
            in_specs=(
                [pl.BlockSpec((BLK, 32), lambda i: (i, 0))
                 for _ in range(S_in)] + [
                    pl.BlockSpec((1, d), lambda i: (0, 0)),
                    pl.BlockSpec((BLK, 1), lambda i: (i, 0)),
                ]),
            out_specs=[
                pl.BlockSpec((BLK, d), lambda i: (i, 0)),
                pl.BlockSpec((1, 1, d), lambda i: (i, 0, 0)),
                pl.BlockSpec((1, 1, d), lambda i: (i, 0, 0)),
            ],
            out_shape=[
                jax.ShapeDtypeStruct((NN, d), jnp.float32),
                jax.ShapeDtypeStruct((GRID, 1, d), jnp.float32),
                jax.ShapeDtypeStruct((GRID, 1, d), jnp.float32),
            ],
        )(*a_list, b.reshape(1, d), dinv)

    return call


def _make_bnapply(d, S_out, d_out, with_pre):
    def body(z_ref, s_ref, q_ref, g_ref, be_ref, *rest):
        m = jnp.sum(s_ref[...], axis=0) * (1.0 / NN)
        v = jnp.sum(q_ref[...], axis=0) * (1.0 / NN) - m * m
        inv = lax.rsqrt(v + 1e-5)
        h = _gelu_f((z_ref[...] - m) * inv * g_ref[...] + be_ref[...])
        if with_pre:
            w_ref, dv_ref = rest[:2]
            o_refs = rest[2:]
            u = jnp.dot(h, w_ref[...],
                        preferred_element_type=jnp.float32) * dv_ref[...]
            _write_slices(u, o_refs)
        else:
            (o_ref,) = rest
            o_ref[...] = h

    def call(z, sums, sq, g, be, W=None, dinv=None):
        in_specs = [
            pl.BlockSpec((BLK, d), lambda i: (i, 0)),
            pl.BlockSpec((GRID, 1, d), lambda i: (0, 0, 0)),
            pl.BlockSpec((GRID, 1, d), lambda i: (0, 0, 0)),
            pl.BlockSpec((1, d), lambda i: (0, 0)),
            pl.BlockSpec((1, d), lambda i: (0, 0)),
        ]
        args = [z, sums, sq, g.reshape(1, d), be.reshape(1, d)]
        if with_pre:
            in_specs += [
                pl.BlockSpec((d, d_out), lambda i: (0, 0)),
                pl.BlockSpec((BLK, 1), lambda i: (i, 0)),
            ]
            args += [W, dinv]
            out_specs, out_shape = _slice_outspecs(S_out)
        else:
            out_specs = pl.BlockSpec((BLK, d), lambda i: (i, 0))
            out_shape = jax.ShapeDtypeStruct((NN, d), jnp.float32)
        return pl.pallas_call(
            body,
            grid=(GRID,),
            in_specs=in_specs,
            out_specs=out_specs,
            out_shape=out_shape,
        )(*args)

    return call


def _head_body(p_ref, w1_ref, b1_ref, w2_ref, b2_ref, o_ref):
    pooled = jnp.max(p_ref[...], axis=0)
    t = jnp.dot(pooled, w1_ref[...],
                preferred_element_type=jnp.float32) + b1_ref[...]
    o_ref[...] = jnp.dot(t, w2_ref[...],
                         preferred_element_type=jnp.float32) + b2_ref[...]


def _head_call(parts, lin1_W, lin1_b, lin_W, lin_b):
    return pl.pallas_call(
        _head_body,
        out_shape=jax.ShapeDtypeStruct((NGROUPS, 2), jnp.float32),
    )(parts, lin1_W, lin1_b.reshape(1, 10), lin_W, lin_b.reshape(1, 2))


_bnstat64 = _make_bnstat(2, 64)
_bnstat128 = _make_bnstat(4, 128)
_bnapply64_pre = _make_bnapply(64, 2, 64, True)
_bnapply128_pre = _make_bnapply(128, 4, 128, True)
_bnapply64_out = _make_bnapply(64, 0, 0, False)
_post64_128 = _make_post_pre(2, 4, 64, 128)
_post128_64 = _make_post_pre(4, 2, 128, 64)


def kernel(x, ei, batch, W1, b1, W2, b2, W3, b3, W4, b4, W5, b5, g1, be1, g2,
           be2, g3, be3, lin1_W, lin1_b, lin_W, lin_b):
    f32 = jnp.float32
    src = ei[0]
    dst = ei[1]
    pad_idx = jnp.arange(EPAD - EE, dtype=jnp.int32) % 128
    src_p = jnp.concatenate([src, pad_idx]).reshape(EROWS, 128)
    dst_p = jnp.concatenate([dst, pad_idx + NN]).reshape(EROWS, 128)

    deg_parts = _deg_call(dst_p)
    dinv = _dinv_call(deg_parts)

    u1 = _pre1_call(x.astype(f32), W1, dinv)
    a1 = _spmm2(*u1, src_p, dst_p)
    z1, s1, q1 = _bnstat64(a1, b1, dinv)
    u2 = _bnapply64_pre(z1, s1, q1, g1, be1, W2, dinv)
    a2 = _spmm2(*u2, src_p, dst_p)
    u3 = _post64_128(a2, b2, dinv, W3)
    a3 = _spmm4(*u3, src_p, dst_p)
    z3, s3, q3 = _bnstat128(a3, b3, dinv)
    u4 = _bnapply128_pre(z3, s3, q3, g2, be2, W4, dinv)
    a4 = _spmm4(*u4, src_p, dst_p)
    u5 = _post128_64(a4, b4, dinv, W5)
    a5 = _spmm2(*u5, src_p, dst_p)
    z5, s5, q5 = _bnstat64(a5, b5, dinv)
    h5 = _bnapply64_out(z5, s5, q5, g3, be3)

    parts = _pool_call(h5, batch)
    return _head_call(parts, lin1_W, lin1_b, lin_W, lin_b)


# speedup vs baseline: 1.1511x; 1.0260x over previous
import jax
import jax.numpy as jnp
from jax import lax
from jax.experimental import pallas as pl
from jax.experimental.pallas import tpu as pltpu
from jax.experimental.pallas import tpu_sc as plsc

NN = 50000
EE = 800000
EPAD = 819200
EROWS = EPAD // 128
NPAD = NN + 128
NGROUPS = 64
BLK = 5000
GRID = NN // BLK
RPT = 3128
RPT_LAST = NN - 15 * RPT


def _row_split_copy(s, src_fn, dst_fn):
    @pl.when(s < 15)
    def _():
        pltpu.sync_copy(src_fn(RPT), dst_fn(RPT))

    @pl.when(s == 15)
    def _():
        pltpu.sync_copy(src_fn(RPT_LAST), dst_fn(RPT_LAST))


_SC_MESH = dict(core_axis_name="c", subcore_axis_name="s")


def _deg_body(dst2_hbm, out_hbm, ones_v, didx_v, acc_s):
    c = lax.axis_index("c")
    s = lax.axis_index("s")

    def fill(i, carry):
        ones_v[i, pl.ds(0, 16)] = jnp.full((16,), 1.0, jnp.float32)
        return carry

    lax.fori_loop(0, RPT, fill, 0, unroll=4)
    r0 = s * RPT
    _row_split_copy(s, lambda n: ones_v.at[pl.ds(0, n)],
                    lambda n: acc_s.at[pl.ds(r0, n)])
    plsc.subcore_barrier()

    def blk(b, carry):
        er0 = c * 3200 + s * 200 + b * 8
        pltpu.sync_copy(dst2_hbm.at[pl.ds(er0, 8)], didx_v)
        for j in range(8):
            pltpu.sync_copy(ones_v.at[pl.ds(0, 128)], acc_s.at[didx_v.at[j]],
                            add=True)
        return carry

    lax.fori_loop(0, 25, blk, 0)
    plsc.subcore_barrier()
    _row_split_copy(s, lambda n: acc_s.at[pl.ds(r0, n)],
                    lambda n: out_hbm.at[pl.ds(c * NN + r0, n)])


_deg_call = pl.kernel(
    _deg_body,
    out_type=jax.ShapeDtypeStruct((2 * NN, 16), jnp.float32),
    mesh=plsc.VectorSubcoreMesh(**_SC_MESH),
    compiler_params=pltpu.CompilerParams(use_tc_tiling_on_sc=False),
    scratch_types=[
        pltpu.VMEM((RPT, 16), jnp.float32),
        pltpu.VMEM((8, 128), jnp.int32),
        pltpu.VMEM_SHARED((NPAD, 16), jnp.float32),
    ],
)


def _make_spmm(S):
    S_pc = S // 2

    def body(*refs):
        us = refs[:S]
        src2_hbm = refs[S]
        dst2_hbm = refs[S + 1]
        outs = refs[S + 3:2 * S + 3]
        sidx_v, didx_v, rows_v, isem, sems, acc_s = refs[2 * S + 3:]
        c = lax.axis_index("c")
        s = lax.axis_index("s")

        def slice_body(u_hbm, out_hbm):
            r0 = s * RPT
            _row_split_copy(s, lambda n: u_hbm.at[pl.ds(r0, n)],
                            lambda n: acc_s.at[pl.ds(r0, n)])
            plsc.subcore_barrier()

            row0 = s * 400
            pltpu.sync_copy(src2_hbm.at[pl.ds(row0, 16)],
                            sidx_v.at[pl.ds(0, 16)])
            pltpu.sync_copy(dst2_hbm.at[pl.ds(row0, 16)],
                            didx_v.at[pl.ds(0, 16)])

            def blk(b, bcarry):
                cur = lax.rem(b, 2) * 16
                nxt = lax.rem(b + 1, 2) * 16

                @pl.when(b > 0)
                def _():
                    pltpu.make_async_copy(
                        src2_hbm.at[pl.ds(row0 + b * 16, 16)],
                        sidx_v.at[pl.ds(cur, 16)], isem).wait()
                    pltpu.make_async_copy(
                        dst2_hbm.at[pl.ds(row0 + b * 16, 16)],
                        didx_v.at[pl.ds(cur, 16)], isem).wait()

                @pl.when(b < 24)
                def _():
                    pltpu.async_copy(
                        src2_hbm.at[pl.ds(row0 + (b + 1) * 16, 16)],
                        sidx_v.at[pl.ds(nxt, 16)], isem)
                    pltpu.async_copy(
                        dst2_hbm.at[pl.ds(row0 + (b + 1) * 16, 16)],
                        didx_v.at[pl.ds(nxt, 16)], isem)

                cps = [
                    pltpu.async_copy(u_hbm.at[sidx_v.at[cur + j]],
                                     rows_v.at[j], sems.at[j])
                    for j in range(5)
                ]
                for j in range(16):
                    jb = j % 5
                    cps[jb].wait()
                    pltpu.sync_copy(rows_v.at[jb],
                                    acc_s.at[didx_v.at[cur + j]], add=True)
                    if j + 5 < 16:
                        cps[jb] = pltpu.async_copy(
                            u_hbm.at[sidx_v.at[cur + j + 5]], rows_v.at[jb],
                            sems.at[jb])
                return bcarry

            lax.fori_loop(0, 25, blk, 0)
            plsc.subcore_barrier()
            _row_split_copy(s, lambda n: acc_s.at[pl.ds(r0, n)],
                            lambda n: out_hbm.at[pl.ds(r0, n)])
            plsc.subcore_barrier()

        for sl in range(S):
            owner = sl // S_pc

            @pl.when(c == owner)
            def _(u_ref=us[sl], o_ref=outs[sl]):
                slice_body(u_ref, o_ref)

    return pl.kernel(
        body,
        out_type=[jax.ShapeDtypeStruct((NN, 32), jnp.float32)
                  for _ in range(S)],
        mesh=plsc.VectorSubcoreMesh(**_SC_MESH),
        compiler_params=pltpu.CompilerParams(use_tc_tiling_on_sc=False),
        scratch_types=[
            pltpu.VMEM((32, 128), jnp.int32),
            pltpu.VMEM((32, 128), jnp.int32),
            pltpu.VMEM((5, 128, 32), jnp.float32),
            pltpu.SemaphoreType.DMA,
            pltpu.SemaphoreType.DMA((5,)),
            pltpu.VMEM_SHARED((NPAD, 32), jnp.float32),
        ],
    )


_spmm2 = _make_spmm(2)

RPW = 1568
RBLK = 224
RPW_LAST = NN - 31 * RPW
POOL_TAIL = RPW_LAST - 6 * RBLK


def _pool_body(h_hbm, b_hbm, out_hbm, rows_v, bid_v, pacc_v):
    c = lax.axis_index("c")
    s = lax.axis_index("s")
    w = s * 2 + c

    def initp(i, carry):
        for j in range(4):
            pacc_v[i, pl.ds(j * 16, 16)] = jnp.full((16,), -jnp.inf,
                                                    jnp.float32)
        return carry

    lax.fori_loop(0, NGROUPS, initp, 0, unroll=4)

    def do_block(bb, nrows):
        pltpu.sync_copy(h_hbm.at[pl.ds(w * RPW + bb * RBLK, nrows)],
                        rows_v.at[pl.ds(0, nrows)])

        def row(r, rcarry):
            gid = bid_v[pl.ds(bb * RBLK + r, 16)][0]
            for j in range(4):
                sl_ = pl.ds(j * 16, 16)
                pacc_v[gid, sl_] = jnp.maximum(pacc_v[gid, sl_],
                                               rows_v[r, sl_])
            return rcarry

        lax.fori_loop(0, nrows, row, 0)

    @pl.when(w < 31)
    def _():
        pltpu.sync_copy(b_hbm.at[pl.ds(w * RPW, RPW)],
                        bid_v.at[pl.ds(0, RPW)])

        def blk(bb, carry):
            do_block(bb, RBLK)
            return carry

        lax.fori_loop(0, 7, blk, 0)

    @pl.when(w == 31)
    def _():
        pltpu.sync_copy(b_hbm.at[pl.ds(w * RPW, RPW_LAST)],
                        bid_v.at[pl.ds(0, RPW_LAST)])

        def blk(bb, carry):
            do_block(bb, RBLK)
            return carry

        lax.fori_loop(0, 6, blk, 0)
        do_block(6, POOL_TAIL)

    pltpu.sync_copy(pacc_v, out_hbm.at[w])


_pool_call = pl.kernel(
    _pool_body,
    out_type=jax.ShapeDtypeStruct((32, NGROUPS, 64), jnp.float32),
    mesh=plsc.VectorSubcoreMesh(**_SC_MESH),
    compiler_params=pltpu.CompilerParams(use_tc_tiling_on_sc=False),
    scratch_types=[
        pltpu.VMEM((RBLK, 64), jnp.float32),
        pltpu.VMEM((RPW + 16,), jnp.int32),
        pltpu.VMEM((NGROUPS, 64), jnp.float32),
    ],
)


def _gelu_f(x):
    return x * 0.5 * (1.0 + lax.erf(x * 0.7071067811865476))


def _dinv_body(d0_ref, d1_ref, o_ref):
    d = d0_ref[:, 0:1] + d1_ref[:, 0:1] - 1.0
    o_ref[...] = lax.rsqrt(d)


def _dinv_call(deg_parts):
    nb = NN // BLK
    return pl.pallas_call(
        _dinv_body,
        grid=(GRID,),
        in_specs=[
            pl.BlockSpec((BLK, 16), lambda i: (i, 0)),
            pl.BlockSpec((BLK, 16), lambda i: (nb + i, 0)),
        ],
        out_specs=pl.BlockSpec((BLK, 1), lambda i: (i, 0)),
        out_shape=jax.ShapeDtypeStruct((NN, 1), jnp.float32),
    )(deg_parts, deg_parts)


def _slice_outspecs(S_out):
    specs = [pl.BlockSpec((BLK, 32), lambda i: (i, 0)) for _ in range(S_out)]
    shapes = [jax.ShapeDtypeStruct((NN, 32), jnp.float32)
              for _ in range(S_out)]
    return specs, shapes


def _write_slices(u, o_refs):
    for i, o in enumerate(o_refs):
        o[...] = u[:, 32 * i:32 * (i + 1)]


def _pre1_body(x_ref, w_ref, dv_ref, *o_refs):
    u = jnp.dot(x_ref[...], w_ref[...],
                preferred_element_type=jnp.float32) * dv_ref[...]
    _write_slices(u, o_refs)


def _pre1_call(x, W1, dinv):
    o_specs, o_shapes = _slice_outspecs(2)
    return pl.pallas_call(
        _pre1_body,
        grid=(GRID,),
        in_specs=[
            pl.BlockSpec((BLK, 2), lambda i: (i, 0)),
            pl.BlockSpec((2, 64), lambda i: (0, 0)),
            pl.BlockSpec((BLK, 1), lambda i: (i, 0)),
        ],
        out_specs=o_specs,
        out_shape=o_shapes,
    )(x, W1, dinv)


def _make_post_pre(S_in, S_out, d_in, d_out):
    def body(*refs):
        a_refs = refs[:S_in]
        b_ref, dv_ref, w_ref = refs[S_in:S_in + 3]
        o_refs = refs[S_in + 3:]
        acc = jnp.concatenate([a[...] for a in a_refs], axis=1)
        dv = dv_ref[...]
        h = _gelu_f(dv * acc + b_ref[...])
        u = jnp.dot(h, w_ref[...], preferred_element_type=jnp.float32) * dv
        _write_slices(u, o_refs)

    def call(a_list, b, dinv, W):
        o_specs, o_shapes = _slice_outspecs(S_out)
        return pl.pallas_call(
            body,
            grid=(GRID,),
            in_specs=(
                [pl.BlockSpec((BLK, 32), lambda i: (i, 0))
                 for _ in range(S_in)] + [
                    pl.BlockSpec((1, d_in), lambda i: (0, 0)),
                    pl.BlockSpec((BLK, 1), lambda i: (i, 0)),
                    pl.BlockSpec((d_in, d_out), lambda i: (0, 0)),
                ]),
            out_specs=o_specs,
            out_shape=o_shapes,
        )(*a_list, b.reshape(1, d_in), dinv, W)

    return call


def _make_bnstat(S_in, d):
    def body(*refs):
        a_refs = refs[:S_in]
        b_ref, dv_ref = refs[S_in:S_in + 2]
        z_ref, s_ref, q_ref = refs[S_in + 2:]
        acc = jnp.concatenate([a[...] for a in a_refs], axis=1)
        z = dv_ref[...] * acc + b_ref[...]
        z_ref[...] = z
        s_ref[0] = jnp.sum(z, axis=0, keepdims=True)
        q_ref[0] = jnp.sum(z * z, axis=0, keepdims=True)

    def call(a_list, b, dinv):
        return pl.pallas_call(
            body,
            grid=(GRID,),
            in_specs=(
                [pl.BlockSpec((BLK, 32), lambda i: (i, 0))
                 for _ in range(S_in)] + [
                    pl.BlockSpec((1, d), lambda i: (0, 0)),
                    pl.BlockSpec((BLK, 1), lambda i: (i, 0)),
                ]),
            out_specs=[
                pl.BlockSpec((BLK, d), lambda i: (i, 0)),
                pl.BlockSpec((1, 1, d), lambda i: (i, 0, 0)),
                pl.BlockSpec((1, 1, d), lambda i: (i, 0, 0)),
            ],
            out_shape=[
                jax.ShapeDtypeStruct((NN, d), jnp.float32),
                jax.ShapeDtypeStruct((GRID, 1, d), jnp.float32),
                jax.ShapeDtypeStruct((GRID, 1, d), jnp.float32),
            ],
        )(*a_list, b.reshape(1, d), dinv)

    return call


def _make_bnapply(d, S_out, d_out, with_pre):
    def body(z_ref, s_ref, q_ref, g_ref, be_ref, *rest):
        m = jnp.sum(s_ref[...], axis=0) * (1.0 / NN)
        v = jnp.sum(q_ref[...], axis=0) * (1.0 / NN) - m * m
        inv = lax.rsqrt(v + 1e-5)
        h = _gelu_f((z_ref[...] - m) * inv * g_ref[...] + be_ref[...])
        if with_pre:
            w_ref, dv_ref = rest[:2]
            o_refs = rest[2:]
            u = jnp.dot(h, w_ref[...],
                        preferred_element_type=jnp.float32) * dv_ref[...]
            _write_slices(u, o_refs)
        else:
            (o_ref,) = rest
            o_ref[...] = h

    def call(z, sums, sq, g, be, W=None, dinv=None):
        in_specs = [
            pl.BlockSpec((BLK, d), lambda i: (i, 0)),
            pl.BlockSpec((GRID, 1, d), lambda i: (0, 0, 0)),
            pl.BlockSpec((GRID, 1, d), lambda i: (0, 0, 0)),
            pl.BlockSpec((1, d), lambda i: (0, 0)),
            pl.BlockSpec((1, d), lambda i: (0, 0)),
        ]
        args = [z, sums, sq, g.reshape(1, d), be.reshape(1, d)]
        if with_pre:
            in_specs += [
                pl.BlockSpec((d, d_out), lambda i: (0, 0)),
                pl.BlockSpec((BLK, 1), lambda i: (i, 0)),
            ]
            args += [W, dinv]
            out_specs, out_shape = _slice_outspecs(S_out)
        else:
            out_specs = pl.BlockSpec((BLK, d), lambda i: (i, 0))
            out_shape = jax.ShapeDtypeStruct((NN, d), jnp.float32)
        return pl.pallas_call(
            body,
            grid=(GRID,),
            in_specs=in_specs,
            out_specs=out_specs,
            out_shape=out_shape,
        )(*args)

    return call


def _bn_h(z_ref, s_ref, q_ref, g_ref, be_ref):
    m = jnp.sum(s_ref[...], axis=0) * (1.0 / NN)
    v = jnp.sum(q_ref[...], axis=0) * (1.0 / NN) - m * m
    inv = lax.rsqrt(v + 1e-5)
    return _gelu_f((z_ref[...] - m) * inv * g_ref[...] + be_ref[...])


def _bn_base_specs(d):
    return [
        pl.BlockSpec((BLK, d), lambda i: (i, 0)),
        pl.BlockSpec((GRID, 1, d), lambda i: (0, 0, 0)),
        pl.BlockSpec((GRID, 1, d), lambda i: (0, 0, 0)),
        pl.BlockSpec((1, d), lambda i: (0, 0)),
        pl.BlockSpec((1, d), lambda i: (0, 0)),
    ]


def _make_bnapply_partial(d_in, d_out):
    def body(z_ref, s_ref, q_ref, g_ref, be_ref, w_ref, o_ref):
        h = _bn_h(z_ref, s_ref, q_ref, g_ref, be_ref)
        o_ref[...] = jnp.dot(h, w_ref[...],
                             preferred_element_type=jnp.float32)

    def call(z, sums, sq, g, be, W):
        return pl.pallas_call(
            body,
            grid=(GRID,),
            in_specs=_bn_base_specs(d_in) + [
                pl.BlockSpec((d_in, d_out), lambda i: (0, 0)),
            ],
            out_specs=pl.BlockSpec((BLK, d_out), lambda i: (i, 0)),
            out_shape=jax.ShapeDtypeStruct((NN, d_out), jnp.float32),
        )(z, sums, sq, g.reshape(1, d_in), be.reshape(1, d_in), W)

    return call


def _make_bnapply_combine(d_in, S_out, d_out):
    def body(z_ref, s_ref, q_ref, g_ref, be_ref, w_ref, p_ref, dv_ref,
             *o_refs):
        h = _bn_h(z_ref, s_ref, q_ref, g_ref, be_ref)
        u = (p_ref[...] + jnp.dot(h, w_ref[...],
                                  preferred_element_type=jnp.float32)
             ) * dv_ref[...]
        _write_slices(u, o_refs)

    def call(z, sums, sq, g, be, W, p, dinv):
        o_specs, o_shapes = _slice_outspecs(S_out)
        return pl.pallas_call(
            body,
            grid=(GRID,),
            in_specs=_bn_base_specs(d_in) + [
                pl.BlockSpec((d_in, d_out), lambda i: (0, 0)),
                pl.BlockSpec((BLK, d_out), lambda i: (i, 0)),
                pl.BlockSpec((BLK, 1), lambda i: (i, 0)),
            ],
            out_specs=o_specs,
            out_shape=o_shapes,
        )(z, sums, sq, g.reshape(1, d_in), be.reshape(1, d_in), W, p, dinv)

    return call


def _make_post_partial(S_in, d_in, d_out):
    def body(*refs):
        a_refs = refs[:S_in]
        b_ref, dv_ref, w_ref, o_ref = refs[S_in:]
        acc = jnp.concatenate([a[...] for a in a_refs], axis=1)
        h = _gelu_f(dv_ref[...] * acc + b_ref[...])
        o_ref[...] = jnp.dot(h, w_ref[...],
                             preferred_element_type=jnp.float32)

    def call(a_list, b, dinv, W):
        return pl.pallas_call(
            body,
            grid=(GRID,),
            in_specs=(
                [pl.BlockSpec((BLK, 32), lambda i: (i, 0))
                 for _ in range(S_in)] + [
                    pl.BlockSpec((1, d_in), lambda i: (0, 0)),
                    pl.BlockSpec((BLK, 1), lambda i: (i, 0)),
                    pl.BlockSpec((d_in, d_out), lambda i: (0, 0)),
                ]),
            out_specs=pl.BlockSpec((BLK, d_out), lambda i: (i, 0)),
            out_shape=jax.ShapeDtypeStruct((NN, d_out), jnp.float32),
        )(*a_list, b.reshape(1, d_in), dinv, W)

    return call


def _make_post_combine(S_in, S_out, d_in, d_out):
    def body(*refs):
        a_refs = refs[:S_in]
        b_ref, dv_ref, w_ref, p_ref = refs[S_in:S_in + 4]
        o_refs = refs[S_in + 4:]
        acc = jnp.concatenate([a[...] for a in a_refs], axis=1)
        dv = dv_ref[...]
        h = _gelu_f(dv * acc + b_ref[...])
        u = (p_ref[...] + jnp.dot(h, w_ref[...],
                                  preferred_element_type=jnp.float32)) * dv
        _write_slices(u, o_refs)

    def call(a_list, b, dinv, W, p):
        o_specs, o_shapes = _slice_outspecs(S_out)
        return pl.pallas_call(
            body,
            grid=(GRID,),
            in_specs=(
                [pl.BlockSpec((BLK, 32), lambda i: (i, 0))
                 for _ in range(S_in)] + [
                    pl.BlockSpec((1, d_in), lambda i: (0, 0)),
                    pl.BlockSpec((BLK, 1), lambda i: (i, 0)),
                    pl.BlockSpec((d_in, d_out), lambda i: (0, 0)),
                    pl.BlockSpec((BLK, d_out), lambda i: (i, 0)),
                ]),
            out_specs=o_specs,
            out_shape=o_shapes,
        )(*a_list, b.reshape(1, d_in), dinv, W, p)

    return call


def _head_body(p_ref, w1_ref, b1_ref, w2_ref, b2_ref, o_ref):
    pooled = jnp.max(p_ref[...], axis=0)
    t = jnp.dot(pooled, w1_ref[...],
                preferred_element_type=jnp.float32) + b1_ref[...]
    o_ref[...] = jnp.dot(t, w2_ref[...],
                         preferred_element_type=jnp.float32) + b2_ref[...]


def _head_call(parts, lin1_W, lin1_b, lin_W, lin_b):
    return pl.pallas_call(
        _head_body,
        out_shape=jax.ShapeDtypeStruct((NGROUPS, 2), jnp.float32),
    )(parts, lin1_W, lin1_b.reshape(1, 10), lin_W, lin_b.reshape(1, 2))


_bnstat64 = _make_bnstat(2, 64)
_bnapply64_pre = _make_bnapply(64, 2, 64, True)
_bnapply64_out = _make_bnapply(64, 0, 0, False)
_post64_128 = _make_post_pre(2, 4, 64, 128)
_bnp_partial64_128 = _make_bnapply_partial(64, 128)
_bnp_combine64_128 = _make_bnapply_combine(64, 4, 128)
_post_partial64_64 = _make_post_partial(2, 64, 64)
_post_combine64_64 = _make_post_combine(2, 2, 64, 64)


def kernel(x, ei, batch, W1, b1, W2, b2, W3, b3, W4, b4, W5, b5, g1, be1, g2,
           be2, g3, be3, lin1_W, lin1_b, lin_W, lin_b):
    f32 = jnp.float32
    src = ei[0]
    dst = ei[1]
    pad_idx = jnp.arange(EPAD - EE, dtype=jnp.int32) % 128
    src_p = jnp.concatenate([src, pad_idx]).reshape(EROWS, 128)
    dst_p = jnp.concatenate([dst, pad_idx + NN]).reshape(EROWS, 128)

    deg_parts = _deg_call(dst_p)
    dinv = _dinv_call(deg_parts)

    u1 = _pre1_call(x.astype(f32), W1, dinv)
    a1 = _spmm2(*u1, src_p, dst_p, u1[0])
    z1, s1, q1 = _bnstat64(a1, b1, dinv)
    u2 = _bnapply64_pre(z1, s1, q1, g1, be1, W2, dinv)
    a2 = _spmm2(*u2, src_p, dst_p, u2[0])
    u3 = _post64_128(a2, b2, dinv, W3)
    a3f = _spmm2(u3[0], u3[1], src_p, dst_p, u3[2])
    a3s = _spmm2(u3[2], u3[3], src_p, dst_p, a3f[0])
    z3a, s3a, q3a = _bnstat64(a3f, b3[:64], dinv)
    p4 = _bnp_partial64_128(z3a, s3a, q3a, g2[:64], be2[:64], W4[:64])
    z3b, s3b, q3b = _bnstat64(a3s, b3[64:], dinv)
    u4 = _bnp_combine64_128(z3b, s3b, q3b, g2[64:], be2[64:], W4[64:], p4,
                            dinv)
    a4f = _spmm2(u4[0], u4[1], src_p, dst_p, u4[2])
    a4s = _spmm2(u4[2], u4[3], src_p, dst_p, a4f[0])
    p5 = _post_partial64_64(a4f, b4[:64], dinv, W5[:64])
    u5 = _post_combine64_64(a4s, b4[64:], dinv, W5[64:], p5)
    a5 = _spmm2(*u5, src_p, dst_p, u5[0])
    z5, s5, q5 = _bnstat64(a5, b5, dinv)
    h5 = _bnapply64_out(z5, s5, q5, g3, be3)

    parts = _pool_call(h5, batch)
    return _head_call(parts, lin1_W, lin1_b, lin_W, lin_b)
